# baseline (speedup 1.0000x reference)
import functools

import jax
import jax.numpy as jnp
from jax import lax
from jax.experimental import pallas as pl
from jax.experimental.pallas import tpu as pltpu
from jax.experimental.pallas import tpu_sc as plsc

NC = 2
NS = 16
LANES = 16
SPLIT0 = 0.405


def _spmm_partials(h, src, dst, w, chunk):
    n, d = h.shape
    e = src.shape[0]
    assert chunk % LANES == 0 and chunk <= 128
    quantum = 3 * chunk
    per_pair = -(-e // NS)
    e0 = max(quantum, int(per_pair * SPLIT0) // quantum * quantum)
    e1 = max(quantum, -(-(per_pair - e0) // quantum) * quantum)
    e_pad = NS * (e0 + e1) - e
    k0 = e0 // chunk
    k1 = e1 // chunk
    e_stage = max(e0, e1)
    rpt = ((n // NS + 7) // 8) * 8
    rpt_last = n - (NS - 1) * rpt
    mesh = plsc.VectorSubcoreMesh(core_axis_name="c", subcore_axis_name="s")

    @functools.partial(
        pl.kernel,
        out_type=jax.ShapeDtypeStruct((NC, n, d), jnp.float32),
        mesh=mesh,
        scratch_types=[
            pltpu.VMEM((e_stage,), jnp.int32),
            pltpu.VMEM((chunk,), jnp.int32),
            pltpu.VMEM((chunk,), jnp.int32),
            pltpu.VMEM((chunk,), jnp.int32),
            pltpu.VMEM((chunk,), jnp.float32),
            pltpu.VMEM((chunk,), jnp.float32),
            pltpu.VMEM((chunk,), jnp.float32),
            pltpu.VMEM((chunk, d), jnp.float32),
            pltpu.VMEM((chunk, d), jnp.float32),
            pltpu.VMEM((chunk, d), jnp.float32),
            pltpu.VMEM_SHARED((n, d), jnp.float32),
            pltpu.SemaphoreType.DMA,
            pltpu.SemaphoreType.DMA,
            pltpu.SemaphoreType.DMA,
            pltpu.SemaphoreType.DMA,
            pltpu.SemaphoreType.DMA,
            pltpu.SemaphoreType.DMA,
            pltpu.SemaphoreType.DMA,
            pltpu.SemaphoreType.DMA,
            pltpu.SemaphoreType.DMA,
        ],
    )
    def spmm_kernel(h_hbm, src_hbm, dst_hbm, w_hbm, zero_hbm, out_hbm,
                    src_v, dbuf0, dbuf1, dbuf2, wbuf0, wbuf1, wbuf2,
                    rows0, rows1, rows2, acc,
                    gsem0, gsem1, gsem2, ssem0, ssem1, ssem2,
                    fsem0, fsem1, fsem2):
        cid = lax.axis_index("c")
        sid = lax.axis_index("s")

        r0 = sid * rpt

        @pl.when(sid < NS - 1)
        def _():
            pltpu.sync_copy(zero_hbm.at[pl.ds(r0, rpt)],
                            acc.at[pl.ds(r0, rpt)])

        @pl.when(sid == NS - 1)
        def _():
            pltpu.sync_copy(zero_hbm.at[pl.ds(r0, rpt_last)],
                            acc.at[pl.ds(r0, rpt_last)])

        plsc.subcore_barrier()

        rows = (rows0, rows1, rows2)
        dbufs = (dbuf0, dbuf1, dbuf2)
        wbufs = (wbuf0, wbuf1, wbuf2)
        gsems = (gsem0, gsem1, gsem2)
        ssems = (ssem0, ssem1, ssem2)
        fsems = (fsem0, fsem1, fsem2)

        base = jnp.where(cid == 0, sid * e0, NS * e0 + sid * e1)
        n_chunks = jnp.where(cid == 0, k0, k1)

        @pl.when(cid == 0)
        def _():
            pltpu.sync_copy(src_hbm.at[pl.ds(base, e0)],
                            src_v.at[pl.ds(0, e0)])

        @pl.when(cid == 1)
        def _():
            pltpu.sync_copy(src_hbm.at[pl.ds(base, e1)],
                            src_v.at[pl.ds(0, e1)])

        def fetch(i, b):
            off = base + i * chunk
            pltpu.async_copy(dst_hbm.at[pl.ds(off, chunk)], dbufs[b], fsems[b])
            pltpu.async_copy(w_hbm.at[pl.ds(off, chunk)], wbufs[b], fsems[b])

        def fetch_wait(b):
            pltpu.make_async_copy(dst_hbm.at[pl.ds(0, chunk)], dbufs[b],
                                  fsems[b]).wait()
            pltpu.make_async_copy(w_hbm.at[pl.ds(0, chunk)], wbufs[b],
                                  fsems[b]).wait()

        def gather(i, b):
            pltpu.async_copy(h_hbm.at[src_v.at[pl.ds(i * chunk, chunk)]],
                             rows[b], gsems[b])

        def gather_wait(b):
            pltpu.make_async_copy(h_hbm.at[src_v.at[pl.ds(0, chunk)]],
                                  rows[b], gsems[b]).wait()

        def scatter_wait(b):
            pltpu.make_async_copy(rows[b], acc.at[dbufs[b]], ssems[b]).wait()

        def scale(rb, wb):
            def group(g, c2):
                wv = wb[pl.ds(g * LANES, LANES)]
                for k in range(LANES):
                    wk = wv[k]
                    row = g * LANES + k
                    for j in range(d // LANES):
                        sl = pl.ds(j * LANES, LANES)
                        rb[row, sl] = rb[row, sl] * wk
                return c2

            lax.fori_loop(0, chunk // LANES, group, 0)

        fetch(0, 0)
        fetch(1, 1)
        gather(0, 0)
        gather(1, 1)

        def triple_body(i3, carry):
            for b in range(3):
                i = i3 * 3 + b
                gather_wait(b)
                fetch_wait(b)
                scale(rows[b], wbufs[b])
                pltpu.async_copy(rows[b], acc.at[dbufs[b]], ssems[b], add=True)
                bn = (b + 2) % 3
                if b == 0:
                    @pl.when(i3 >= 1)
                    def _():
                        scatter_wait(bn)

                    fetch(i + 2, bn)
                    gather(i + 2, bn)
                else:
                    @pl.when(i + 2 < n_chunks)
                    def _():
                        scatter_wait(bn)
                        fetch(i + 2, bn)
                        gather(i + 2, bn)
            return carry

        lax.fori_loop(0, n_chunks // 3, triple_body, 0)

        for b in range(3):
            scatter_wait(b)

        plsc.subcore_barrier()

        @pl.when(sid < NS - 1)
        def _():
            pltpu.sync_copy(acc.at[pl.ds(r0, rpt)],
                            out_hbm.at[cid].at[pl.ds(r0, rpt)])

        @pl.when(sid == NS - 1)
        def _():
            pltpu.sync_copy(acc.at[pl.ds(r0, rpt_last)],
                            out_hbm.at[cid].at[pl.ds(r0, rpt_last)])

    zero = jnp.zeros((n, d), jnp.float32)
    src1 = jnp.pad(src, (0, e_pad))
    dst1 = jnp.pad(dst, (0, e_pad))
    w1 = jnp.pad(w, (0, e_pad))
    return spmm_kernel(h, src1, dst1, w1, zero)


def _linear_tc(x, w, b, bm):
    n, d = x.shape

    def body(x_ref, w_ref, b_ref, o_ref):
        o_ref[...] = (
            jnp.dot(x_ref[...], w_ref[...], preferred_element_type=jnp.float32)
            + b_ref[...]
        )

    return pl.pallas_call(
        body,
        grid=(n // bm,),
        in_specs=[
            pl.BlockSpec((bm, d), lambda i: (i, 0)),
            pl.BlockSpec((d, d), lambda i: (0, 0)),
            pl.BlockSpec((1, d), lambda i: (0, 0)),
        ],
        out_specs=pl.BlockSpec((bm, d), lambda i: (i, 0)),
        out_shape=jax.ShapeDtypeStruct((n, d), jnp.float32),
    )(x, w, b.reshape(1, d))


def _combine_relu_linear_tc(p, w, b, bm):
    _, n, d = p.shape

    def body(p_ref, w_ref, b_ref, o_ref):
        h = jnp.maximum(p_ref[0] + p_ref[1], 0.0)
        o_ref[...] = (
            jnp.dot(h, w_ref[...], preferred_element_type=jnp.float32)
            + b_ref[...]
        )

    return pl.pallas_call(
        body,
        grid=(n // bm,),
        in_specs=[
            pl.BlockSpec((NC, bm, d), lambda i: (0, i, 0)),
            pl.BlockSpec((d, d), lambda i: (0, 0)),
            pl.BlockSpec((1, d), lambda i: (0, 0)),
        ],
        out_specs=pl.BlockSpec((bm, d), lambda i: (i, 0)),
        out_shape=jax.ShapeDtypeStruct((n, d), jnp.float32),
    )(p, w, b.reshape(1, d))


def _combine_tc(p, bm):
    _, n, d = p.shape

    def body(p_ref, o_ref):
        o_ref[...] = p_ref[0] + p_ref[1]

    return pl.pallas_call(
        body,
        grid=(n // bm,),
        in_specs=[pl.BlockSpec((NC, bm, d), lambda i: (0, i, 0))],
        out_specs=pl.BlockSpec((bm, d), lambda i: (i, 0)),
        out_shape=jax.ShapeDtypeStruct((n, d), jnp.float32),
    )(p)


def kernel(x, edge_weight, W0, b0, W1, b1, edge_index):
    dst = edge_index[0]
    src = edge_index[1]
    bm = 1000
    chunk = 80

    h0 = _linear_tc(x, W0, b0, bm)
    p1 = _spmm_partials(h0, src, dst, edge_weight, chunk)
    h1 = _combine_relu_linear_tc(p1, W1, b1, bm)
    p2 = _spmm_partials(h1, src, dst, edge_weight, chunk)
    return _combine_tc(p2, bm)


# speedup vs baseline: 1.1837x; 1.1837x over previous
import functools

import jax
import jax.numpy as jnp
from jax import lax
from jax.experimental import pallas as pl
from jax.experimental.pallas import tpu as pltpu
from jax.experimental.pallas import tpu_sc as plsc

NC = 2
NS = 16
LANES = 16
SPLIT0 = 0.68


def _spmm_partials(h, src, dst, w, chunk):
    n, d = h.shape
    e = src.shape[0]
    assert chunk % LANES == 0 and chunk <= 128
    quantum = 3 * chunk
    per_pair = -(-e // NS)
    e0 = max(quantum, int(per_pair * SPLIT0) // quantum * quantum)
    e1 = max(quantum, -(-(per_pair - e0) // quantum) * quantum)
    e_pad = NS * (e0 + e1) - e
    k0 = e0 // chunk
    k1 = e1 // chunk
    e_stage = max(e0, e1)
    rpt = ((n // NS + 7) // 8) * 8
    rpt_last = n - (NS - 1) * rpt
    mesh = plsc.VectorSubcoreMesh(core_axis_name="c", subcore_axis_name="s")

    @functools.partial(
        pl.kernel,
        out_type=jax.ShapeDtypeStruct((NC, n, d), jnp.float32),
        mesh=mesh,
        scratch_types=[
            pltpu.VMEM((e_stage,), jnp.int32),
            pltpu.VMEM((chunk,), jnp.int32),
            pltpu.VMEM((chunk,), jnp.int32),
            pltpu.VMEM((chunk,), jnp.int32),
            pltpu.VMEM((chunk,), jnp.float32),
            pltpu.VMEM((chunk,), jnp.float32),
            pltpu.VMEM((chunk,), jnp.float32),
            pltpu.VMEM((chunk, d), jnp.float32),
            pltpu.VMEM((chunk, d), jnp.float32),
            pltpu.VMEM((chunk, d), jnp.float32),
            pltpu.VMEM_SHARED((n, d), jnp.float32),
            pltpu.SemaphoreType.DMA,
            pltpu.SemaphoreType.DMA,
            pltpu.SemaphoreType.DMA,
            pltpu.SemaphoreType.DMA,
            pltpu.SemaphoreType.DMA,
            pltpu.SemaphoreType.DMA,
            pltpu.SemaphoreType.DMA,
            pltpu.SemaphoreType.DMA,
            pltpu.SemaphoreType.DMA,
        ],
    )
    def spmm_kernel(h_hbm, src_hbm, dst_hbm, w_hbm, zero_hbm, out_hbm,
                    src_v, dbuf0, dbuf1, dbuf2, wbuf0, wbuf1, wbuf2,
                    rows0, rows1, rows2, acc,
                    gsem0, gsem1, gsem2, ssem0, ssem1, ssem2,
                    fsem0, fsem1, fsem2):
        cid = lax.axis_index("c")
        sid = lax.axis_index("s")

        r0 = sid * rpt

        @pl.when(sid < NS - 1)
        def _():
            pltpu.sync_copy(zero_hbm.at[pl.ds(r0, rpt)],
                            acc.at[pl.ds(r0, rpt)])

        @pl.when(sid == NS - 1)
        def _():
            pltpu.sync_copy(zero_hbm.at[pl.ds(r0, rpt_last)],
                            acc.at[pl.ds(r0, rpt_last)])

        plsc.subcore_barrier()

        rows = (rows0, rows1, rows2)
        dbufs = (dbuf0, dbuf1, dbuf2)
        wbufs = (wbuf0, wbuf1, wbuf2)
        gsems = (gsem0, gsem1, gsem2)
        ssems = (ssem0, ssem1, ssem2)
        fsems = (fsem0, fsem1, fsem2)

        base = jnp.where(cid == 0, sid * e0, NS * e0 + sid * e1)
        n_chunks = jnp.where(cid == 0, k0, k1)

        @pl.when(cid == 0)
        def _():
            pltpu.sync_copy(src_hbm.at[pl.ds(base, e0)],
                            src_v.at[pl.ds(0, e0)])

        @pl.when(cid == 1)
        def _():
            pltpu.sync_copy(src_hbm.at[pl.ds(base, e1)],
                            src_v.at[pl.ds(0, e1)])

        def fetch(i, b):
            off = base + i * chunk
            pltpu.async_copy(dst_hbm.at[pl.ds(off, chunk)], dbufs[b], fsems[b])
            pltpu.async_copy(w_hbm.at[pl.ds(off, chunk)], wbufs[b], fsems[b])

        def fetch_wait(b):
            pltpu.make_async_copy(dst_hbm.at[pl.ds(0, chunk)], dbufs[b],
                                  fsems[b]).wait()
            pltpu.make_async_copy(w_hbm.at[pl.ds(0, chunk)], wbufs[b],
                                  fsems[b]).wait()

        def gather(i, b):
            pltpu.async_copy(h_hbm.at[src_v.at[pl.ds(i * chunk, chunk)]],
                             rows[b], gsems[b])

        def gather_wait(b):
            pltpu.make_async_copy(h_hbm.at[src_v.at[pl.ds(0, chunk)]],
                                  rows[b], gsems[b]).wait()

        def scatter_wait(b):
            pltpu.make_async_copy(rows[b], acc.at[dbufs[b]], ssems[b]).wait()

        def scale(rb, wb):
            def group(g, c2):
                wv = wb[pl.ds(g * LANES, LANES)]
                for k in range(LANES):
                    wk = wv[k]
                    row = g * LANES + k
                    for j in range(d // LANES):
                        sl = pl.ds(j * LANES, LANES)
                        rb[row, sl] = rb[row, sl] * wk
                return c2

            lax.fori_loop(0, chunk // LANES, group, 0)

        fetch(0, 0)
        fetch(1, 1)
        gather(0, 0)
        gather(1, 1)

        def triple_body(i3, carry):
            for b in range(3):
                i = i3 * 3 + b
                gather_wait(b)
                fetch_wait(b)
                scale(rows[b], wbufs[b])
                pltpu.async_copy(rows[b], acc.at[dbufs[b]], ssems[b], add=True)
                bn = (b + 2) % 3
                if b == 0:
                    @pl.when(i3 >= 1)
                    def _():
                        scatter_wait(bn)

                    fetch(i + 2, bn)
                    gather(i + 2, bn)
                else:
                    @pl.when(i + 2 < n_chunks)
                    def _():
                        scatter_wait(bn)
                        fetch(i + 2, bn)
                        gather(i + 2, bn)
            return carry

        lax.fori_loop(0, n_chunks // 3, triple_body, 0)

        for b in range(3):
            scatter_wait(b)

        plsc.subcore_barrier()

        @pl.when(sid < NS - 1)
        def _():
            pltpu.sync_copy(acc.at[pl.ds(r0, rpt)],
                            out_hbm.at[cid].at[pl.ds(r0, rpt)])

        @pl.when(sid == NS - 1)
        def _():
            pltpu.sync_copy(acc.at[pl.ds(r0, rpt_last)],
                            out_hbm.at[cid].at[pl.ds(r0, rpt_last)])

    zero = jnp.zeros((n, d), jnp.float32)
    src1 = jnp.pad(src, (0, e_pad))
    dst1 = jnp.pad(dst, (0, e_pad))
    w1 = jnp.pad(w, (0, e_pad))
    return spmm_kernel(h, src1, dst1, w1, zero)


def _linear_tc(x, w, b, bm):
    n, d = x.shape

    def body(x_ref, w_ref, b_ref, o_ref):
        o_ref[...] = (
            jnp.dot(x_ref[...], w_ref[...], preferred_element_type=jnp.float32)
            + b_ref[...]
        )

    return pl.pallas_call(
        body,
        grid=(n // bm,),
        in_specs=[
            pl.BlockSpec((bm, d), lambda i: (i, 0)),
            pl.BlockSpec((d, d), lambda i: (0, 0)),
            pl.BlockSpec((1, d), lambda i: (0, 0)),
        ],
        out_specs=pl.BlockSpec((bm, d), lambda i: (i, 0)),
        out_shape=jax.ShapeDtypeStruct((n, d), jnp.float32),
    )(x, w, b.reshape(1, d))


def _combine_relu_linear_tc(p, w, b, bm):
    _, n, d = p.shape

    def body(p_ref, w_ref, b_ref, o_ref):
        h = jnp.maximum(p_ref[0] + p_ref[1], 0.0)
        o_ref[...] = (
            jnp.dot(h, w_ref[...], preferred_element_type=jnp.float32)
            + b_ref[...]
        )

    return pl.pallas_call(
        body,
        grid=(n // bm,),
        in_specs=[
            pl.BlockSpec((NC, bm, d), lambda i: (0, i, 0)),
            pl.BlockSpec((d, d), lambda i: (0, 0)),
            pl.BlockSpec((1, d), lambda i: (0, 0)),
        ],
        out_specs=pl.BlockSpec((bm, d), lambda i: (i, 0)),
        out_shape=jax.ShapeDtypeStruct((n, d), jnp.float32),
    )(p, w, b.reshape(1, d))


def _combine_tc(p, bm):
    _, n, d = p.shape

    def body(p_ref, o_ref):
        o_ref[...] = p_ref[0] + p_ref[1]

    return pl.pallas_call(
        body,
        grid=(n // bm,),
        in_specs=[pl.BlockSpec((NC, bm, d), lambda i: (0, i, 0))],
        out_specs=pl.BlockSpec((bm, d), lambda i: (i, 0)),
        out_shape=jax.ShapeDtypeStruct((n, d), jnp.float32),
    )(p)


def kernel(x, edge_weight, W0, b0, W1, b1, edge_index):
    dst = edge_index[0]
    src = edge_index[1]
    bm = 1000
    chunk = 80

    h0 = _linear_tc(x, W0, b0, bm)
    p1 = _spmm_partials(h0, src, dst, edge_weight, chunk)
    h1 = _combine_relu_linear_tc(p1, W1, b1, bm)
    p2 = _spmm_partials(h1, src, dst, edge_weight, chunk)
    return _combine_tc(p2, bm)
